# baseline (speedup 1.0000x reference)
import jax
import jax.numpy as jnp
from jax.experimental import pallas as pl

B, HW, DIM = 64, 1024, 64
K = 9 * 114
KPAD = 1152
T = B * HW
TB = 1024


def _vq_body(z_ref, at_ref, a_ref, o_ref):
    zb = z_ref[0]
    nrm = jnp.sqrt(jnp.sum(zb * zb, axis=-1, keepdims=True))
    zn = zb / jnp.maximum(nrm, 1e-12)
    sims = jnp.dot(zn.astype(jnp.bfloat16), at_ref[...],
                   preferred_element_type=jnp.float32)
    idx = jnp.argmax(sims, axis=-1)
    col = jax.lax.broadcasted_iota(jnp.int32, (TB, KPAD), 1)
    onehot = (col == idx[:, None]).astype(jnp.bfloat16)
    o_ref[0] = jnp.dot(onehot, a_ref[...],
                       preferred_element_type=jnp.float32)


def kernel(z, attractors):
    A = attractors.reshape(-1, DIM)
    a_pad = jnp.concatenate(
        [A, jnp.broadcast_to(A[:1], (KPAD - K, DIM))], axis=0)
    at_pad = a_pad.T.astype(jnp.bfloat16)
    a_bf = a_pad.astype(jnp.bfloat16)
    out = pl.pallas_call(
        _vq_body,
        grid=(B,),
        in_specs=[
            pl.BlockSpec((1, HW, DIM), lambda i: (i, 0, 0)),
            pl.BlockSpec((DIM, KPAD), lambda i: (0, 0)),
            pl.BlockSpec((KPAD, DIM), lambda i: (0, 0)),
        ],
        out_specs=pl.BlockSpec((1, HW, DIM), lambda i: (i, 0, 0)),
        out_shape=jax.ShapeDtypeStruct((B, HW, DIM), jnp.float32),
    )(z, at_pad, a_bf)
    return out


# speedup vs baseline: 1.0827x; 1.0827x over previous
import functools

import jax
import jax.numpy as jnp
from jax import lax
from jax.experimental import pallas as pl
from jax.experimental.pallas import tpu as pltpu
from jax.experimental.pallas import tpu_sc as plsc

B, HW, DIM = 64, 1024, 64
K = 9 * 114
KPAD = 1152
T = B * HW
TB = 4096
NB = T // TB

NW = 32
RPW = T // NW
CH = 128


def _idx_body(z_ref, at_ref, o_ref):
    zb = z_ref[...]
    nrm = jnp.sqrt(jnp.sum(zb * zb, axis=-1, keepdims=True))
    zn = zb / jnp.maximum(nrm, 1e-12)
    sims = jnp.dot(zn.astype(jnp.bfloat16), at_ref[...],
                   preferred_element_type=jnp.float32)
    o_ref[0, 0] = (KPAD - 1) - jnp.argmax(sims, axis=-1).astype(jnp.int32)


def _gather_body(a_hbm, idx_hbm, out_hbm, idx_v, rows_v, sem):
    wid = lax.axis_index("s") * 2 + lax.axis_index("c")
    base = wid * RPW
    pltpu.sync_copy(idx_hbm.at[pl.ds(base, RPW)], idx_v)

    def chunk(j, carry):
        off = j * CH
        pltpu.async_copy(a_hbm.at[idx_v.at[pl.ds(off, CH)]], rows_v,
                         sem).wait()
        pltpu.sync_copy(rows_v, out_hbm.at[pl.ds(base + off, CH)])
        return carry

    lax.fori_loop(0, RPW // CH, chunk, 0)


_sc_gather = functools.partial(
    pl.kernel,
    mesh=plsc.VectorSubcoreMesh(core_axis_name="c", subcore_axis_name="s"),
    out_type=jax.ShapeDtypeStruct((T, DIM), jnp.float32),
    scratch_types=[
        pltpu.VMEM((RPW,), jnp.int32),
        pltpu.VMEM((CH, DIM), jnp.float32),
        pltpu.SemaphoreType.DMA,
    ],
    compiler_params=pltpu.CompilerParams(use_tc_tiling_on_sc=False),
)(_gather_body)


def kernel(z, attractors):
    A = attractors.reshape(-1, DIM)
    a_pad = jnp.concatenate(
        [A, jnp.broadcast_to(A[:1], (KPAD - K, DIM))], axis=0)
    at_pad = a_pad.T[:, ::-1].astype(jnp.bfloat16)
    zf = z.reshape(T, DIM)
    idx = pl.pallas_call(
        _idx_body,
        grid=(NB,),
        in_specs=[
            pl.BlockSpec((TB, DIM), lambda i: (i, 0)),
            pl.BlockSpec((DIM, KPAD), lambda i: (0, 0)),
        ],
        out_specs=pl.BlockSpec((1, 1, TB), lambda i: (i, 0, 0)),
        out_shape=jax.ShapeDtypeStruct((NB, 1, TB), jnp.int32),
    )(zf, at_pad)
    out = _sc_gather(A, idx.reshape(T))
    return out.reshape(B, HW, DIM)


# speedup vs baseline: 1.7435x; 1.6103x over previous
import functools

import jax
import jax.numpy as jnp
from jax import lax
from jax.experimental import pallas as pl
from jax.experimental.pallas import tpu as pltpu
from jax.experimental.pallas import tpu_sc as plsc

B, HW, DIM = 64, 1024, 64
K = 9 * 114
KPAD = 1152
T = B * HW
TB = 4096
NB = T // TB

NW = 32
RPW = T // NW
CH = 128


def _idx_body(zt_ref, a_ref, o_ref):
    zt = zt_ref[...]
    nrm = jnp.sqrt(jnp.sum(zt * zt, axis=0, keepdims=True))
    zn = zt / jnp.maximum(nrm, 1e-12)
    simsT = jnp.dot(a_ref[...], zn.astype(jnp.bfloat16),
                    preferred_element_type=jnp.float32)
    o_ref[0, 0] = (KPAD - 1) - jnp.argmax(simsT, axis=0).astype(jnp.int32)


def _gather_body(a_hbm, idx_hbm, out_hbm, idx_v, rows_v, sem):
    wid = lax.axis_index("s") * 2 + lax.axis_index("c")
    base = wid * RPW
    pltpu.sync_copy(idx_hbm.at[pl.ds(base, RPW)], idx_v)

    def chunk(j, carry):
        off = j * CH
        pltpu.async_copy(a_hbm.at[idx_v.at[pl.ds(off, CH)]], rows_v,
                         sem).wait()
        pltpu.sync_copy(rows_v, out_hbm.at[pl.ds(base + off, CH)])
        return carry

    lax.fori_loop(0, RPW // CH, chunk, 0)


_sc_gather = functools.partial(
    pl.kernel,
    mesh=plsc.VectorSubcoreMesh(core_axis_name="c", subcore_axis_name="s"),
    out_type=jax.ShapeDtypeStruct((T, DIM), jnp.float32),
    scratch_types=[
        pltpu.VMEM((RPW,), jnp.int32),
        pltpu.VMEM((CH, DIM), jnp.float32),
        pltpu.SemaphoreType.DMA,
    ],
    compiler_params=pltpu.CompilerParams(use_tc_tiling_on_sc=False),
)(_gather_body)


def kernel(z, attractors):
    A = attractors.reshape(-1, DIM)
    a_pad = jnp.concatenate(
        [A, jnp.broadcast_to(A[:1], (KPAD - K, DIM))], axis=0)
    a_rev = a_pad[::-1].astype(jnp.bfloat16)
    zt = z.reshape(T, DIM).T
    idx = pl.pallas_call(
        _idx_body,
        grid=(NB,),
        in_specs=[
            pl.BlockSpec((DIM, TB), lambda i: (0, i)),
            pl.BlockSpec((KPAD, DIM), lambda i: (0, 0)),
        ],
        out_specs=pl.BlockSpec((1, 1, TB), lambda i: (i, 0, 0)),
        out_shape=jax.ShapeDtypeStruct((NB, 1, TB), jnp.int32),
    )(zt, a_rev)
    out = _sc_gather(A, idx.reshape(T))
    return out.reshape(B, HW, DIM)


# speedup vs baseline: 1.8347x; 1.0523x over previous
import functools

import jax
import jax.numpy as jnp
from jax import lax
from jax.experimental import pallas as pl
from jax.experimental.pallas import tpu as pltpu
from jax.experimental.pallas import tpu_sc as plsc

B, HW, DIM = 64, 1024, 64
K = 9 * 114
KPAD = 1152
T = B * HW
TB = 4096
NB = T // TB

NW = 32
RPW = T // NW
CH = 128


def _idx_body(zt_ref, a_ref, o_ref):
    zt = zt_ref[...]
    nrm = jnp.sqrt(jnp.sum(zt * zt, axis=0, keepdims=True))
    zn = zt / jnp.maximum(nrm, 1e-12)
    simsT = jnp.dot(a_ref[...], zn.astype(jnp.bfloat16),
                    preferred_element_type=jnp.float32)
    o_ref[0, 0] = (KPAD - 1) - jnp.argmax(simsT, axis=0).astype(jnp.int32)


def _gather_body(a_hbm, idx_hbm, out_hbm, idx_v, rows_v, sem):
    wid = lax.axis_index("s") * 2 + lax.axis_index("c")
    base = wid * RPW
    pltpu.sync_copy(idx_hbm.at[pl.ds(base, RPW)], idx_v)

    def chunk(j, carry):
        off = j * CH
        pltpu.async_copy(a_hbm.at[idx_v.at[pl.ds(off, CH)]], rows_v,
                         sem).wait()
        pltpu.sync_copy(rows_v, out_hbm.at[pl.ds(base + off, CH)])
        return carry

    lax.fori_loop(0, RPW // CH, chunk, 0)


_sc_gather = functools.partial(
    pl.kernel,
    mesh=plsc.VectorSubcoreMesh(core_axis_name="c", subcore_axis_name="s"),
    out_type=jax.ShapeDtypeStruct((T, 128), jnp.float32),
    scratch_types=[
        pltpu.VMEM((RPW,), jnp.int32),
        pltpu.VMEM((CH, 128), jnp.float32),
        pltpu.SemaphoreType.DMA,
    ],
)(_gather_body)


def kernel(z, attractors):
    A = attractors.reshape(-1, DIM)
    a_pad = jnp.concatenate(
        [A, jnp.broadcast_to(A[:1], (KPAD - K, DIM))], axis=0)
    a_rev = a_pad[::-1].astype(jnp.bfloat16)
    zt = z.reshape(T, DIM).T
    idx = pl.pallas_call(
        _idx_body,
        grid=(NB,),
        in_specs=[
            pl.BlockSpec((DIM, TB), lambda i: (0, i)),
            pl.BlockSpec((KPAD, DIM), lambda i: (0, 0)),
        ],
        out_specs=pl.BlockSpec((1, 1, TB), lambda i: (i, 0, 0)),
        out_shape=jax.ShapeDtypeStruct((NB, 1, TB), jnp.int32),
    )(zt, a_rev)
    a_wide = jnp.pad(a_pad, ((0, 0), (0, 128 - DIM)))
    out = _sc_gather(a_wide, idx.reshape(T))
    return out[:, :DIM].reshape(B, HW, DIM)
